# baseline (speedup 1.0000x reference)
import functools

import jax
import jax.numpy as jnp
import numpy as np
from jax import lax
from jax.experimental import pallas as pl
from jax.experimental.pallas import tpu as pltpu
from jax.experimental.pallas import tpu_sc as plsc

VOCAB = 30522
HIDDEN = 768
MAX_POS = 512
BATCH = 256
SEQ = 512
EPS = 1e-12

L = 16
HG = HIDDEN // (2 * L)
NC = 2
NS = 16
NW = NC * NS
B_PER_W = BATCH // NW
C = 32
NP = SEQ // C
NT = NP * B_PER_W
NBUF = 2

_GDN = lax.GatherDimensionNumbers(
    offset_dims=(), collapsed_slice_dims=(0,), start_index_map=(0,)
)


def _hsum(x):
    lanes = lax.iota(jnp.int32, L)
    for m in (8, 4, 2, 1):
        perm = lax.bitwise_xor(lanes, m)
        x = x + lax.gather(
            x, perm[:, None], _GDN, slice_sizes=(1,),
            mode=lax.GatherScatterMode.PROMISE_IN_BOUNDS,
        )
    return x


def _vrsqrt(v):
    i = plsc.bitcast(v, jnp.int32)
    y = plsc.bitcast(jnp.int32(0x5F3759DF) - (i >> 1), jnp.float32)
    for _ in range(2):
        y = y * (1.5 - 0.5 * v * y * y)
    return y


def _unpack2(v):
    lo = plsc.bitcast(v << 16, jnp.float32)
    hi = plsc.bitcast(v & jnp.int32(-65536), jnp.float32)
    return lo, hi


def _body(ids, tts, wemb, pemb, temb, gam, bet, out,
          idxs, ttvs, wbs, ybs, pbuf, gbuf, bbuf, t0buf, t1buf, svb, rsb,
          gsems, osems):
    wid = lax.axis_index("s") * NC + lax.axis_index("c")

    pltpu.sync_copy(gam, gbuf)
    pltpu.sync_copy(bet, bbuf)
    pltpu.sync_copy(temb.at[0], t0buf)
    pltpu.sync_copy(temb.at[1], t1buf)

    def tile_dst(t):
        p = t // B_PER_W
        b = wid * B_PER_W + (t % B_PER_W)
        return out.at[b, pl.ds(p * C, C)]

    def issue(t, r):
        @pl.when(t < NT)
        def _():
            p = t // B_PER_W
            b = wid * B_PER_W + (t % B_PER_W)
            pltpu.sync_copy(ids.at[b, pl.ds(p * C, C)], idxs[r])
            pltpu.async_copy(tts.at[b, pl.ds(p * C, C)], ttvs[r].at[pl.ds(0, C)],
                             gsems[r])
            pltpu.async_copy(wemb.at[idxs[r]], wbs[r], gsems[r])

    def _pack2(a0, a1):
        i0 = plsc.bitcast(a0, jnp.int32) + 32768
        i1 = plsc.bitcast(a1, jnp.int32) + 32768
        return lax.shift_right_logical(i0, 16) | (i1 & jnp.int32(-65536))

    def reload_pbuf(p):
        pltpu.sync_copy(pemb.at[pl.ds(p * C, C)], pbuf.at[pl.ds(0, C)])

        def fold(j, _):
            for g in range(HG):
                sl0 = pl.ds(g * 2 * L, L)
                sl1 = pl.ds(g * 2 * L + L, L)
                p0, p1 = _unpack2(pbuf[j, pl.ds(g * L, L)])
                pbuf[C + j, pl.ds(g * L, L)] = _pack2(p0 + t1buf[sl0],
                                                      p1 + t1buf[sl1])
                pbuf[j, pl.ds(g * L, L)] = _pack2(p0 + t0buf[sl0],
                                                  p1 + t0buf[sl1])
            return 0
        lax.fori_loop(0, C, fold, 0)

    def compute(t, r):
        wb = wbs[r]
        yb = ybs[r]

        @pl.when(t < NT)
        def _():
            p = t // B_PER_W
            bi = t % B_PER_W
            b = wid * B_PER_W + bi

            @pl.when(bi == 0)
            def _():
                reload_pbuf(p)

            pltpu.make_async_copy(tts.at[b, pl.ds(p * C, C)],
                                  ttvs[r].at[pl.ds(0, C)], gsems[r]).wait()
            pltpu.make_async_copy(wemb.at[idxs[r]], wb, gsems[r]).wait()

            @pl.when(t >= NBUF)
            def _():
                pltpu.make_async_copy(yb, tile_dst(t), osems[r]).wait()

            @plsc.parallel_loop(0, C, 1)
            def _stats(j):
                lanes = lax.iota(jnp.int32, L)
                ttrow = ttvs[r][pl.ds(j, L)]
                ttb = lax.gather(
                    ttrow, (lanes * 0)[:, None], _GDN, slice_sizes=(1,),
                    mode=lax.GatherScatterMode.PROMISE_IN_BOUNDS,
                )
                rowv = ttb * C + j
                z = jnp.zeros((L,), jnp.float32)
                s = [z] * 4
                q = [z] * 4
                for g in range(HG):
                    w0, w1 = _unpack2(wb[j, pl.ds(g * L, L)])
                    p0, p1 = _unpack2(
                        plsc.load_gather(pbuf, [rowv, lanes + g * L]))
                    sl0 = pl.ds(g * 2 * L, L)
                    sl1 = pl.ds(g * 2 * L + L, L)
                    x0 = w0 + p0
                    x1 = w1 + p1
                    yb[j, sl0] = x0
                    yb[j, sl1] = x1
                    s[g & 1] = s[g & 1] + x0
                    q[g & 1] = q[g & 1] + x0 * x0
                    s[2 + (g & 1)] = s[2 + (g & 1)] + x1
                    q[2 + (g & 1)] = q[2 + (g & 1)] + x1 * x1
                sv = _hsum((s[0] + s[1]) + (s[2] + s[3])) * (1.0 / HIDDEN)
                qv = _hsum((q[0] + q[1]) + (q[2] + q[3])) * (1.0 / HIDDEN)
                rs = _vrsqrt(qv - sv * sv + EPS)
                svb[j] = sv
                rsb[j] = rs

            @plsc.parallel_loop(0, C, 1)
            def _apply(j):
                rs = rsb[j]
                svrs = svb[j] * rs
                for g in range(HG):
                    g0, g1 = _unpack2(gbuf[pl.ds(g * L, L)])
                    b0, b1 = _unpack2(bbuf[pl.ds(g * L, L)])
                    sl0 = pl.ds(g * 2 * L, L)
                    sl1 = pl.ds(g * 2 * L + L, L)
                    yb[j, sl0] = (yb[j, sl0] * rs - svrs) * g0 + b0
                    yb[j, sl1] = (yb[j, sl1] * rs - svrs) * g1 + b1

            pltpu.async_copy(yb, tile_dst(t), osems[r])
            issue(t + NBUF, r)

    for r in range(NBUF):
        issue(r, r)

    def step(m, _):
        for r in range(NBUF):
            compute(NBUF * m + r, r)
        return 0
    lax.fori_loop(0, NT // NBUF, step, 0)

    for r in range(NBUF):
        t_last = NT - NBUF + r
        pltpu.make_async_copy(ybs[r], tile_dst(t_last), osems[r]).wait()


def _ileave(a):
    s = a.shape[:-1]
    return (
        a.reshape(s + (HG, 2, L)).swapaxes(-2, -1).reshape(s + (HIDDEN,))
    )


def kernel(input_ids, token_type_ids, word_emb, pos_emb, type_emb, ln_gamma, ln_beta):
    mesh = plsc.VectorSubcoreMesh(
        core_axis_name="c", subcore_axis_name="s", num_cores=NC, num_subcores=NS
    )

    def body(ids, tts, wemb, pemb, temb, gam, bet, out,
             i0, i1, v0, v1, w0, w1, y0, y1,
             pbuf, gbuf, bbuf, t0buf, t1buf, svb, rsb,
             gs0, gs1, os0, os1):
        _body(ids, tts, wemb, pemb, temb, gam, bet, out,
              [i0, i1], [v0, v1], [w0, w1], [y0, y1],
              pbuf, gbuf, bbuf, t0buf, t1buf, svb, rsb,
              [gs0, gs1], [os0, os1])

    f = pl.kernel(
        body,
        out_type=jax.ShapeDtypeStruct((BATCH, SEQ, HIDDEN), jnp.float32),
        mesh=mesh,
        compiler_params=pltpu.CompilerParams(needs_layout_passes=False),
        scratch_types=[
            pltpu.VMEM((C,), jnp.int32),
            pltpu.VMEM((C,), jnp.int32),
            pltpu.VMEM((C + L,), jnp.int32),
            pltpu.VMEM((C + L,), jnp.int32),
            pltpu.VMEM((C, HIDDEN // 2), jnp.int32),
            pltpu.VMEM((C, HIDDEN // 2), jnp.int32),
            pltpu.VMEM((C, HIDDEN), jnp.float32),
            pltpu.VMEM((C, HIDDEN), jnp.float32),
            pltpu.VMEM((2 * C, HIDDEN // 2), jnp.int32),
            pltpu.VMEM((HIDDEN // 2,), jnp.int32),
            pltpu.VMEM((HIDDEN // 2,), jnp.int32),
            pltpu.VMEM((HIDDEN,), jnp.float32),
            pltpu.VMEM((HIDDEN,), jnp.float32),
            pltpu.VMEM((C, L), jnp.float32),
            pltpu.VMEM((C, L), jnp.float32),
            pltpu.SemaphoreType.DMA,
            pltpu.SemaphoreType.DMA,
            pltpu.SemaphoreType.DMA,
            pltpu.SemaphoreType.DMA,
        ],
    )
    return f(
        input_ids.astype(jnp.int32),
        token_type_ids.astype(jnp.int32),
        lax.bitcast_convert_type(
            _ileave(word_emb).astype(jnp.bfloat16).reshape(VOCAB, HIDDEN // 2, 2),
            jnp.int32,
        ),
        lax.bitcast_convert_type(
            _ileave(pos_emb).astype(jnp.bfloat16).reshape(MAX_POS, HIDDEN // 2, 2),
            jnp.int32,
        ),
        type_emb,
        lax.bitcast_convert_type(
            _ileave(ln_gamma).astype(jnp.bfloat16).reshape(HIDDEN // 2, 2),
            jnp.int32,
        ),
        lax.bitcast_convert_type(
            _ileave(ln_beta).astype(jnp.bfloat16).reshape(HIDDEN // 2, 2),
            jnp.int32,
        ),
    )
